# Initial kernel scaffold; baseline (speedup 1.0000x reference)
import jax
import jax.numpy as jnp
from jax.experimental import pallas as pl


def kernel(X, adj_matrix, weight):
    raise NotImplementedError("write your pallas kernel here")


# speedup vs baseline: 1649.3081x; 1649.3081x over previous
import jax
import jax.numpy as jnp
from jax.experimental import pallas as pl


def _sdhgcn_body(adj_ref, x_ref, w_ref, out_ref):
    adj = adj_ref[...].astype(jnp.float32)
    a = (adj_ref[...] != 0).astype(jnp.float32)
    xw = jnp.dot(x_ref[...], w_ref[...],
                 preferred_element_type=jnp.float32)
    out = jax.lax.dot_general(
        a, xw, dimension_numbers=(((0,), (0,)), ((), ())),
        preferred_element_type=jnp.float32)
    in_degree = jnp.sum(adj, axis=0)
    coeff = jax.lax.rsqrt(jnp.maximum(in_degree, 1.0))
    out_ref[...] = jnp.maximum(out * coeff[:, None], 0.0)


def kernel(X, adj_matrix, weight):
    n, d_out = X.shape[0], weight.shape[1]
    return pl.pallas_call(
        _sdhgcn_body,
        out_shape=jax.ShapeDtypeStruct((n, d_out), jnp.float32),
    )(adj_matrix, X, weight)
